# baseline (speedup 1.0000x reference)
import functools

import jax
import jax.numpy as jnp
import numpy as np
from jax import lax
from jax.experimental import pallas as pl
from jax.experimental.pallas import tpu as pltpu
from jax.experimental.pallas import tpu_sc as plsc

N = 10000
E = 320000
MUL = 16
NUM_RADIAL = 8
HIDDEN = 64
WEIGHT_NUMEL = MUL * MUL

NC = 2
NS = 16
NW = NC * NS

CH = 128
NCHUNK = E // CH
GATHER_TRIPS = -(-NCHUNK // NW)
E_HALF = E // 2
NCH_CORE = E_HALF // CH
SCAT_TRIPS = -(-NCH_CORE // NS)
ROWS_PER_TILE = N // NS

_mesh = plsc.VectorSubcoreMesh(core_axis_name="c", subcore_axis_name="s")
_sc_params = pltpu.CompilerParams(use_tc_tiling_on_sc=False)


@functools.partial(
    pl.kernel,
    mesh=_mesh,
    out_type=jax.ShapeDtypeStruct((E, 128), jnp.float32),
    scratch_types=[
        pltpu.VMEM((CH,), jnp.int32),
        pltpu.VMEM((CH, MUL), jnp.float32),
        pltpu.VMEM((CH, 1), jnp.float32),
        pltpu.SemaphoreType.DMA,
    ],
    compiler_params=_sc_params,
)
def _gather_k(x_hbm, src_hbm, el_hbm, feat_hbm, idx_v, rows_v, el_v, sem):
    wid = lax.axis_index("s") * NC + lax.axis_index("c")

    def body(i, carry):
        j = wid + i * NW

        @pl.when(j < NCHUNK)
        def _():
            base = j * CH
            pltpu.sync_copy(src_hbm.at[pl.ds(base, CH)], idx_v)
            pltpu.sync_copy(el_hbm.at[pl.ds(base, CH)], el_v)
            pltpu.async_copy(x_hbm.at[idx_v], rows_v, sem).wait()
            pltpu.sync_copy(rows_v, feat_hbm.at[pl.ds(base, CH), pl.ds(0, MUL)])
            pltpu.sync_copy(el_v, feat_hbm.at[pl.ds(base, CH), pl.ds(MUL, 1)])

        return carry

    lax.fori_loop(0, GATHER_TRIPS, body, 0)


@functools.partial(
    pl.kernel,
    mesh=_mesh,
    out_type=jax.ShapeDtypeStruct((2 * N, 128), jnp.float32),
    scratch_types=[
        pltpu.VMEM((CH,), jnp.int32),
        pltpu.VMEM((CH, MUL), jnp.float32),
        pltpu.VMEM_SHARED((N, MUL), jnp.float32),
        pltpu.SemaphoreType.DMA,
    ],
    compiler_params=_sc_params,
)
def _scatter_k(m_hbm, dst_hbm, zero_hbm, out_hbm, idx_v, rows_v, acc_sh, sem):
    cid = lax.axis_index("c")
    sid = lax.axis_index("s")
    r0 = sid * ROWS_PER_TILE
    pltpu.sync_copy(zero_hbm.at[pl.ds(r0, ROWS_PER_TILE)],
                    acc_sh.at[pl.ds(r0, ROWS_PER_TILE)])
    plsc.subcore_barrier()

    def body(i, carry):
        j = sid + i * NS

        @pl.when(j < NCH_CORE)
        def _():
            base = cid * E_HALF + j * CH
            pltpu.sync_copy(dst_hbm.at[pl.ds(base, CH)], idx_v)
            pltpu.sync_copy(m_hbm.at[pl.ds(base, CH), pl.ds(0, MUL)], rows_v)
            pltpu.sync_copy(rows_v, acc_sh.at[idx_v], add=True)

        return carry

    lax.fori_loop(0, SCAT_TRIPS, body, 0)
    plsc.subcore_barrier()
    pltpu.sync_copy(acc_sh.at[pl.ds(r0, ROWS_PER_TILE)],
                    out_hbm.at[pl.ds(cid * N + r0, ROWS_PER_TILE), pl.ds(0, MUL)])


_BLK = 2560


def _main_body(feat_ref, ea_ref, w1_ref, w2_ref, o_ref):
    feat = feat_ref[...]
    el = feat[:, MUL:MUL + 1]
    ea = ea_ref[...]
    xj = feat[:, 0:MUL]
    centers = lax.broadcasted_iota(
        jnp.int32, (1, NUM_RADIAL), 1).astype(jnp.float32) * np.float32(5.0 / 7.0)
    d = el - centers
    radial = jnp.exp(-0.5 * d * d)
    w1 = w1_ref[...] * np.float32(1.0 / np.sqrt(NUM_RADIAL))
    h = jnp.dot(radial, w1, preferred_element_type=jnp.float32)
    h = h / (1.0 + jnp.exp(-h))
    w2 = w2_ref[...] * np.float32(1.0 / np.sqrt(HIDDEN))
    wts = jnp.dot(h, w2, preferred_element_type=jnp.float32)

    u_t = lax.broadcasted_iota(jnp.int32, (MUL, WEIGHT_NUMEL), 0)
    c_t = lax.broadcasted_iota(jnp.int32, (MUL, WEIGHT_NUMEL), 1)
    tile_m = jnp.where(c_t % MUL == u_t, 1.0, 0.0).astype(jnp.float32)
    xt = jnp.dot(xj, tile_m, preferred_element_type=jnp.float32)
    p = wts * xt
    r_s = lax.broadcasted_iota(jnp.int32, (WEIGHT_NUMEL, MUL), 0)
    w_s = lax.broadcasted_iota(jnp.int32, (WEIGHT_NUMEL, MUL), 1)
    seg_m = jnp.where(r_s // MUL == w_s, 1.0, 0.0).astype(jnp.float32)
    m = jnp.dot(p, seg_m, preferred_element_type=jnp.float32)
    m = m * ea * np.float32(1.0 / np.sqrt(MUL))
    o_ref[:, 0:MUL] = m


def _tc_main(feat, ea, W1, W2):
    grid = (E // _BLK,)
    return pl.pallas_call(
        _main_body,
        grid=grid,
        in_specs=[
            pl.BlockSpec((_BLK, 128), lambda i: (i, 0)),
            pl.BlockSpec((_BLK, 1), lambda i: (i, 0)),
            pl.BlockSpec((NUM_RADIAL, HIDDEN), lambda i: (0, 0)),
            pl.BlockSpec((HIDDEN, WEIGHT_NUMEL), lambda i: (0, 0)),
        ],
        out_specs=pl.BlockSpec((_BLK, 128), lambda i: (i, 0)),
        out_shape=jax.ShapeDtypeStruct((E, 128), jnp.float32),
    )(feat, ea, W1, W2)


def _comb_body(p_ref, x_ref, wsc_ref, o_ref):
    psum = p_ref[0:N, 0:MUL] + p_ref[N:2 * N, 0:MUL]
    wsc = wsc_ref[...] * np.float32(1.0 / np.sqrt(MUL))
    sc = jnp.dot(x_ref[...], wsc, preferred_element_type=jnp.float32)
    o_ref[...] = psum + sc


def _tc_combine(pfat, x, W_sc):
    return pl.pallas_call(
        _comb_body,
        out_shape=jax.ShapeDtypeStruct((N, MUL), jnp.float32),
    )(pfat, x, W_sc)


def kernel(x, edge_attr, edge_length, edge_src, edge_dst, W1, W2, W_sc):
    src = edge_src.astype(jnp.int32)
    dst = edge_dst.astype(jnp.int32)
    el2 = edge_length.reshape(E, 1)
    feat = _gather_k(x, src, el2)
    mfat = _tc_main(feat, edge_attr, W1, W2)
    zeros = jnp.zeros((N, MUL), dtype=jnp.float32)
    pfat = _scatter_k(mfat, dst, zeros)
    return _tc_combine(pfat, x, W_sc)


# speedup vs baseline: 1.9799x; 1.9799x over previous
import functools

import jax
import jax.numpy as jnp
import numpy as np
from jax import lax
from jax.experimental import pallas as pl
from jax.experimental.pallas import tpu as pltpu
from jax.experimental.pallas import tpu_sc as plsc

N = 10000
E = 320000
MUL = 16
NUM_RADIAL = 8
HIDDEN = 64
WEIGHT_NUMEL = MUL * MUL

NC = 2
NS = 16
NW = NC * NS

CH = 128
NCHUNK = E // CH
GATHER_TRIPS = -(-NCHUNK // NW)
E_HALF = E // 2
NCH_CORE = E_HALF // CH
SCAT_TRIPS = -(-NCH_CORE // NS)
ROWS_PER_TILE = N // NS

_mesh = plsc.VectorSubcoreMesh(core_axis_name="c", subcore_axis_name="s")
_sc_params = pltpu.CompilerParams(use_tc_tiling_on_sc=False)


@functools.partial(
    pl.kernel,
    mesh=_mesh,
    out_type=jax.ShapeDtypeStruct((E, 128), jnp.float32),
    scratch_types=[
        pltpu.VMEM((CH,), jnp.int32),
        pltpu.VMEM((CH, MUL), jnp.float32),
        pltpu.SemaphoreType.DMA,
    ],
    compiler_params=_sc_params,
)
def _gather_k(x_hbm, src_hbm, feat_hbm, idx_v, rows_v, sem):
    wid = lax.axis_index("s") * NC + lax.axis_index("c")

    def body(i, carry):
        j = wid + i * NW

        @pl.when(j < NCHUNK)
        def _():
            base = j * CH
            pltpu.sync_copy(src_hbm.at[pl.ds(base, CH)], idx_v)
            pltpu.async_copy(x_hbm.at[idx_v], rows_v, sem).wait()
            pltpu.sync_copy(rows_v, feat_hbm.at[pl.ds(base, CH), pl.ds(0, MUL)])

        return carry

    lax.fori_loop(0, GATHER_TRIPS, body, 0)


@functools.partial(
    pl.kernel,
    mesh=_mesh,
    out_type=jax.ShapeDtypeStruct((2 * N, 128), jnp.float32),
    scratch_types=[
        pltpu.VMEM((CH,), jnp.int32),
        pltpu.VMEM((CH, MUL), jnp.float32),
        pltpu.VMEM_SHARED((N, MUL), jnp.float32),
        pltpu.SemaphoreType.DMA,
    ],
    compiler_params=_sc_params,
)
def _scatter_k(m_hbm, dst_hbm, zero_hbm, out_hbm, idx_v, rows_v, acc_sh, sem):
    cid = lax.axis_index("c")
    sid = lax.axis_index("s")
    r0 = sid * ROWS_PER_TILE
    pltpu.sync_copy(zero_hbm.at[pl.ds(r0, ROWS_PER_TILE)],
                    acc_sh.at[pl.ds(r0, ROWS_PER_TILE)])
    plsc.subcore_barrier()

    def body(i, carry):
        j = sid + i * NS

        @pl.when(j < NCH_CORE)
        def _():
            base = cid * E_HALF + j * CH
            pltpu.sync_copy(dst_hbm.at[pl.ds(base, CH)], idx_v)
            pltpu.sync_copy(m_hbm.at[pl.ds(base, CH), pl.ds(0, MUL)], rows_v)
            pltpu.sync_copy(rows_v, acc_sh.at[idx_v], add=True)

        return carry

    lax.fori_loop(0, SCAT_TRIPS, body, 0)
    plsc.subcore_barrier()
    pltpu.sync_copy(acc_sh.at[pl.ds(r0, ROWS_PER_TILE)],
                    out_hbm.at[pl.ds(cid * N + r0, ROWS_PER_TILE), pl.ds(0, MUL)])


_BLK = 2560


def _main_body(feat_ref, el_ref, ea_ref, w1_ref, w2_ref, o_ref):
    feat = feat_ref[...]
    el_t = el_ref[...].reshape(1, _BLK)
    ea = ea_ref[...]
    xj = feat[:, 0:MUL]
    centers_t = lax.broadcasted_iota(
        jnp.int32, (NUM_RADIAL, 1), 0).astype(jnp.float32) * np.float32(5.0 / 7.0)
    d_t = el_t - centers_t
    radial_t = jnp.exp(-0.5 * d_t * d_t)
    w1 = w1_ref[...] * np.float32(1.0 / np.sqrt(NUM_RADIAL))
    h_t = jnp.dot(w1.T, radial_t, preferred_element_type=jnp.float32)
    h_t = h_t / (1.0 + jnp.exp(-h_t))
    h = jnp.transpose(h_t)
    w2 = w2_ref[...] * np.float32(1.0 / np.sqrt(HIDDEN))
    wts = jnp.dot(h, w2, preferred_element_type=jnp.float32)

    u_t = lax.broadcasted_iota(jnp.int32, (MUL, WEIGHT_NUMEL), 0)
    c_t = lax.broadcasted_iota(jnp.int32, (MUL, WEIGHT_NUMEL), 1)
    tile_m = jnp.where(c_t % MUL == u_t, 1.0, 0.0).astype(jnp.float32)
    xt = jnp.dot(xj, tile_m, preferred_element_type=jnp.float32)
    p = wts * xt
    r_s = lax.broadcasted_iota(jnp.int32, (WEIGHT_NUMEL, MUL), 0)
    w_s = lax.broadcasted_iota(jnp.int32, (WEIGHT_NUMEL, MUL), 1)
    seg_m = jnp.where(r_s // MUL == w_s, 1.0, 0.0).astype(jnp.float32)
    m = jnp.dot(p, seg_m, preferred_element_type=jnp.float32)
    m = m * ea * np.float32(1.0 / np.sqrt(MUL))
    o_ref[:, 0:MUL] = m


def _tc_main(feat, el3, ea, W1, W2):
    grid = (E // _BLK,)
    return pl.pallas_call(
        _main_body,
        grid=grid,
        in_specs=[
            pl.BlockSpec((_BLK, 128), lambda i: (i, 0)),
            pl.BlockSpec((1, 1, _BLK), lambda i: (i, 0, 0)),
            pl.BlockSpec((_BLK, 1), lambda i: (i, 0)),
            pl.BlockSpec((NUM_RADIAL, HIDDEN), lambda i: (0, 0)),
            pl.BlockSpec((HIDDEN, WEIGHT_NUMEL), lambda i: (0, 0)),
        ],
        out_specs=pl.BlockSpec((_BLK, 128), lambda i: (i, 0)),
        out_shape=jax.ShapeDtypeStruct((E, 128), jnp.float32),
    )(feat, el3, ea, W1, W2)


def _comb_body(p_ref, x_ref, wsc_ref, o_ref):
    psum = p_ref[0:N, 0:MUL] + p_ref[N:2 * N, 0:MUL]
    wsc = wsc_ref[...] * np.float32(1.0 / np.sqrt(MUL))
    sc = jnp.dot(x_ref[...], wsc, preferred_element_type=jnp.float32)
    o_ref[...] = psum + sc


def _tc_combine(pfat, x, W_sc):
    return pl.pallas_call(
        _comb_body,
        out_shape=jax.ShapeDtypeStruct((N, MUL), jnp.float32),
    )(pfat, x, W_sc)


def kernel(x, edge_attr, edge_length, edge_src, edge_dst, W1, W2, W_sc):
    src = edge_src.astype(jnp.int32)
    dst = edge_dst.astype(jnp.int32)
    el3 = edge_length.reshape(E // _BLK, 1, _BLK)
    feat = _gather_k(x, src)
    mfat = _tc_main(feat, el3, edge_attr, W1, W2)
    zeros = jnp.zeros((N, MUL), dtype=jnp.float32)
    pfat = _scatter_k(mfat, dst, zeros)
    return _tc_combine(pfat, x, W_sc)


# speedup vs baseline: 3.0852x; 1.5583x over previous
import functools

import jax
import jax.numpy as jnp
import numpy as np
from jax import lax
from jax.experimental import pallas as pl
from jax.experimental.pallas import tpu as pltpu
from jax.experimental.pallas import tpu_sc as plsc

N = 10000
E = 320000
MUL = 16
NUM_RADIAL = 8
HIDDEN = 64
WEIGHT_NUMEL = MUL * MUL

NC = 2
NS = 16
NW = NC * NS

CH = 128
KB = 8
NCHUNK = E // CH
G_TRIPS = -(-NCHUNK // NW)
G_FULL = (NCHUNK // NW) // KB
E_HALF = E // 2
NCH_CORE = E_HALF // CH
S_TRIPS = -(-NCH_CORE // NS)
S_FULL = (NCH_CORE // NS) // KB
ROWS_PER_TILE = N // NS

_mesh = plsc.VectorSubcoreMesh(core_axis_name="c", subcore_axis_name="s")
_sc_params = pltpu.CompilerParams(use_tc_tiling_on_sc=False)


@functools.partial(
    pl.kernel,
    mesh=_mesh,
    out_type=jax.ShapeDtypeStruct((E, 128), jnp.float32),
    scratch_types=[
        pltpu.VMEM((KB, CH), jnp.int32),
        pltpu.VMEM((KB, CH, MUL), jnp.float32),
        pltpu.SemaphoreType.DMA,
        pltpu.SemaphoreType.DMA,
        pltpu.SemaphoreType.DMA,
    ],
    compiler_params=_sc_params,
)
def _gather_k(x_hbm, src_hbm, feat_hbm, idx_v, rows_v, sem_i, sem_g, sem_w):
    wid = lax.axis_index("s") * NC + lax.axis_index("c")

    def body(i, carry):
        js = [wid + (i * KB + b) * NW for b in range(KB)]
        di = [pltpu.async_copy(src_hbm.at[pl.ds(js[b] * CH, CH)],
                               idx_v.at[b], sem_i) for b in range(KB)]
        for d in di:
            d.wait()
        dg = [pltpu.async_copy(x_hbm.at[idx_v.at[b]], rows_v.at[b], sem_g)
              for b in range(KB)]
        for d in dg:
            d.wait()
        dw = [pltpu.async_copy(rows_v.at[b],
                               feat_hbm.at[pl.ds(js[b] * CH, CH), pl.ds(0, MUL)],
                               sem_w) for b in range(KB)]
        for d in dw:
            d.wait()
        return carry

    lax.fori_loop(0, G_FULL, body, 0)

    def tail(i, carry):
        j = wid + i * NW

        @pl.when(j < NCHUNK)
        def _():
            base = j * CH
            pltpu.sync_copy(src_hbm.at[pl.ds(base, CH)], idx_v.at[0])
            pltpu.async_copy(x_hbm.at[idx_v.at[0]], rows_v.at[0], sem_g).wait()
            pltpu.sync_copy(rows_v.at[0],
                            feat_hbm.at[pl.ds(base, CH), pl.ds(0, MUL)])

        return carry

    lax.fori_loop(G_FULL * KB, G_TRIPS, tail, 0)


@functools.partial(
    pl.kernel,
    mesh=_mesh,
    out_type=jax.ShapeDtypeStruct((2 * N, 128), jnp.float32),
    scratch_types=[
        pltpu.VMEM((KB, CH), jnp.int32),
        pltpu.VMEM((KB, CH, MUL), jnp.float32),
        pltpu.VMEM_SHARED((N, MUL), jnp.float32),
        pltpu.SemaphoreType.DMA,
        pltpu.SemaphoreType.DMA,
        pltpu.SemaphoreType.DMA,
    ],
    compiler_params=_sc_params,
)
def _scatter_k(m_hbm, dst_hbm, zero_hbm, out_hbm, idx_v, rows_v, acc_sh,
               sem_i, sem_m, sem_a):
    cid = lax.axis_index("c")
    sid = lax.axis_index("s")
    r0 = sid * ROWS_PER_TILE
    pltpu.sync_copy(zero_hbm.at[pl.ds(r0, ROWS_PER_TILE)],
                    acc_sh.at[pl.ds(r0, ROWS_PER_TILE)])
    plsc.subcore_barrier()

    def body(i, carry):
        js = [sid + (i * KB + b) * NS for b in range(KB)]
        bases = [cid * E_HALF + js[b] * CH for b in range(KB)]
        di = [pltpu.async_copy(dst_hbm.at[pl.ds(bases[b], CH)],
                               idx_v.at[b], sem_i) for b in range(KB)]
        dm = [pltpu.async_copy(m_hbm.at[pl.ds(bases[b], CH), pl.ds(0, MUL)],
                               rows_v.at[b], sem_m) for b in range(KB)]
        for d in di:
            d.wait()
        for d in dm:
            d.wait()
        da = [pltpu.async_copy(rows_v.at[b], acc_sh.at[idx_v.at[b]], sem_a,
                               add=True) for b in range(KB)]
        for d in da:
            d.wait()
        return carry

    lax.fori_loop(0, S_FULL, body, 0)

    def tail(i, carry):
        j = sid + i * NS

        @pl.when(j < NCH_CORE)
        def _():
            base = cid * E_HALF + j * CH
            pltpu.sync_copy(dst_hbm.at[pl.ds(base, CH)], idx_v.at[0])
            pltpu.sync_copy(m_hbm.at[pl.ds(base, CH), pl.ds(0, MUL)],
                            rows_v.at[0])
            pltpu.sync_copy(rows_v.at[0], acc_sh.at[idx_v.at[0]], add=True)

        return carry

    lax.fori_loop(S_FULL * KB, S_TRIPS, tail, 0)
    plsc.subcore_barrier()
    pltpu.sync_copy(acc_sh.at[pl.ds(r0, ROWS_PER_TILE)],
                    out_hbm.at[pl.ds(cid * N + r0, ROWS_PER_TILE), pl.ds(0, MUL)])


_BLK = 2560


def _main_body(feat_ref, el_ref, ea_ref, w1_ref, w2_ref, o_ref):
    feat = feat_ref[...]
    el_t = el_ref[...].reshape(1, _BLK)
    ea_t = ea_ref[...].reshape(1, _BLK)
    xj = feat[:, 0:MUL]
    centers_t = lax.broadcasted_iota(
        jnp.int32, (NUM_RADIAL, 1), 0).astype(jnp.float32) * np.float32(5.0 / 7.0)
    d_t = el_t - centers_t
    radial_t = jnp.exp(-0.5 * d_t * d_t)
    w1 = w1_ref[...] * np.float32(1.0 / np.sqrt(NUM_RADIAL))
    h_t = jnp.dot(w1.T, radial_t, preferred_element_type=jnp.float32)
    h_t = h_t / (1.0 + jnp.exp(-h_t))
    h_t = h_t * (ea_t * np.float32(1.0 / np.sqrt(MUL)))
    h = jnp.transpose(h_t)
    w2 = w2_ref[...] * np.float32(1.0 / np.sqrt(HIDDEN))
    wts = jnp.dot(h, w2, preferred_element_type=jnp.float32)

    u_t = lax.broadcasted_iota(jnp.int32, (MUL, WEIGHT_NUMEL), 0)
    c_t = lax.broadcasted_iota(jnp.int32, (MUL, WEIGHT_NUMEL), 1)
    tile_m = jnp.where(c_t % MUL == u_t, 1.0, 0.0).astype(jnp.float32)
    xt = jnp.dot(xj, tile_m, preferred_element_type=jnp.float32)
    p = wts * xt
    r_s = lax.broadcasted_iota(jnp.int32, (WEIGHT_NUMEL, MUL), 0)
    w_s = lax.broadcasted_iota(jnp.int32, (WEIGHT_NUMEL, MUL), 1)
    seg_m = jnp.where(r_s // MUL == w_s, 1.0, 0.0).astype(jnp.float32)
    m = jnp.dot(p, seg_m, preferred_element_type=jnp.float32)
    o_ref[:, 0:MUL] = m


def _tc_main(feat, el3, ea3, W1, W2):
    grid = (E // _BLK,)
    return pl.pallas_call(
        _main_body,
        grid=grid,
        in_specs=[
            pl.BlockSpec((_BLK, 128), lambda i: (i, 0)),
            pl.BlockSpec((1, 1, _BLK), lambda i: (i, 0, 0)),
            pl.BlockSpec((1, 1, _BLK), lambda i: (i, 0, 0)),
            pl.BlockSpec((NUM_RADIAL, HIDDEN), lambda i: (0, 0)),
            pl.BlockSpec((HIDDEN, WEIGHT_NUMEL), lambda i: (0, 0)),
        ],
        out_specs=pl.BlockSpec((_BLK, 128), lambda i: (i, 0)),
        out_shape=jax.ShapeDtypeStruct((E, 128), jnp.float32),
    )(feat, el3, ea3, W1, W2)


def _comb_body(p_ref, x_ref, wsc_ref, o_ref):
    psum = p_ref[0:N, 0:MUL] + p_ref[N:2 * N, 0:MUL]
    wsc = wsc_ref[...] * np.float32(1.0 / np.sqrt(MUL))
    sc = jnp.dot(x_ref[...], wsc, preferred_element_type=jnp.float32)
    o_ref[...] = psum + sc


def _tc_combine(pfat, x, W_sc):
    return pl.pallas_call(
        _comb_body,
        out_shape=jax.ShapeDtypeStruct((N, MUL), jnp.float32),
    )(pfat, x, W_sc)


def kernel(x, edge_attr, edge_length, edge_src, edge_dst, W1, W2, W_sc):
    src = edge_src.astype(jnp.int32)
    dst = edge_dst.astype(jnp.int32)
    el3 = edge_length.reshape(E // _BLK, 1, _BLK)
    ea3 = edge_attr.reshape(E // _BLK, 1, _BLK)
    feat = _gather_k(x, src)
    mfat = _tc_main(feat, el3, ea3, W1, W2)
    zeros = jnp.zeros((N, MUL), dtype=jnp.float32)
    pfat = _scatter_k(mfat, dst, zeros)
    return _tc_combine(pfat, x, W_sc)


# speedup vs baseline: 4.1360x; 1.3406x over previous
import functools

import jax
import jax.numpy as jnp
import numpy as np
from jax import lax
from jax.experimental import pallas as pl
from jax.experimental.pallas import tpu as pltpu
from jax.experimental.pallas import tpu_sc as plsc

N = 10000
E = 320000
MUL = 16
NUM_RADIAL = 8
HIDDEN = 64
WEIGHT_NUMEL = MUL * MUL

NC = 2
NS = 16
NW = NC * NS

CH = 128
KB = 8
NCHUNK = E // CH
G_TRIPS = -(-NCHUNK // NW)
G_FULL = (NCHUNK // NW) // KB
E_HALF = E // 2
NCH_CORE = E_HALF // CH
S_TRIPS = -(-NCH_CORE // NS)
S_FULL = (NCH_CORE // NS) // KB
ROWS_PER_TILE = N // NS

_mesh = plsc.VectorSubcoreMesh(core_axis_name="c", subcore_axis_name="s")
_sc_params = pltpu.CompilerParams(use_tc_tiling_on_sc=False)


@functools.partial(
    pl.kernel,
    mesh=_mesh,
    out_type=jax.ShapeDtypeStruct((E, 128), jnp.float32),
    scratch_types=[
        pltpu.VMEM((KB, CH), jnp.int32),
        pltpu.VMEM((KB, CH, MUL), jnp.float32),
        pltpu.SemaphoreType.DMA,
        pltpu.SemaphoreType.DMA,
        pltpu.SemaphoreType.DMA,
    ],
    compiler_params=_sc_params,
)
def _gather_k(x_hbm, src_hbm, feat_hbm, idx_v, rows_v, sem_i, sem_g, sem_w):
    wid = lax.axis_index("s") * NC + lax.axis_index("c")

    def body(i, carry):
        js = [wid + (i * KB + b) * NW for b in range(KB)]
        di = [pltpu.async_copy(src_hbm.at[pl.ds(js[b] * CH, CH)],
                               idx_v.at[b], sem_i) for b in range(KB)]
        for d in di:
            d.wait()
        dg = [pltpu.async_copy(x_hbm.at[idx_v.at[b]], rows_v.at[b], sem_g)
              for b in range(KB)]
        for d in dg:
            d.wait()
        dw = [pltpu.async_copy(rows_v.at[b],
                               feat_hbm.at[pl.ds(js[b] * CH, CH), pl.ds(0, MUL)],
                               sem_w) for b in range(KB)]
        for d in dw:
            d.wait()
        return carry

    lax.fori_loop(0, G_FULL, body, 0)

    def tail(i, carry):
        j = wid + i * NW

        @pl.when(j < NCHUNK)
        def _():
            base = j * CH
            pltpu.sync_copy(src_hbm.at[pl.ds(base, CH)], idx_v.at[0])
            pltpu.async_copy(x_hbm.at[idx_v.at[0]], rows_v.at[0], sem_g).wait()
            pltpu.sync_copy(rows_v.at[0],
                            feat_hbm.at[pl.ds(base, CH), pl.ds(0, MUL)])

        return carry

    lax.fori_loop(G_FULL * KB, G_TRIPS, tail, 0)


@functools.partial(
    pl.kernel,
    mesh=_mesh,
    out_type=jax.ShapeDtypeStruct((2 * N, 128), jnp.float32),
    scratch_types=[
        pltpu.VMEM((KB, CH), jnp.int32),
        pltpu.VMEM((KB, CH, MUL), jnp.float32),
        pltpu.VMEM_SHARED((N, MUL), jnp.float32),
        pltpu.SemaphoreType.DMA,
        pltpu.SemaphoreType.DMA,
        pltpu.SemaphoreType.DMA,
    ],
    compiler_params=_sc_params,
)
def _scatter_k(m_hbm, dst_hbm, zero_hbm, out_hbm, idx_v, rows_v, acc_sh,
               sem_i, sem_m, sem_a):
    cid = lax.axis_index("c")
    sid = lax.axis_index("s")
    r0 = sid * ROWS_PER_TILE
    pltpu.sync_copy(zero_hbm.at[pl.ds(r0, ROWS_PER_TILE)],
                    acc_sh.at[pl.ds(r0, ROWS_PER_TILE)])
    plsc.subcore_barrier()

    def body(i, carry):
        js = [sid + (i * KB + b) * NS for b in range(KB)]
        bases = [cid * E_HALF + js[b] * CH for b in range(KB)]
        di = [pltpu.async_copy(dst_hbm.at[pl.ds(bases[b], CH)],
                               idx_v.at[b], sem_i) for b in range(KB)]
        dm = [pltpu.async_copy(m_hbm.at[pl.ds(bases[b], CH), pl.ds(0, MUL)],
                               rows_v.at[b], sem_m) for b in range(KB)]
        for d in di:
            d.wait()
        for d in dm:
            d.wait()
        da = [pltpu.async_copy(rows_v.at[b], acc_sh.at[idx_v.at[b]], sem_a,
                               add=True) for b in range(KB)]
        for d in da:
            d.wait()
        return carry

    lax.fori_loop(0, S_FULL, body, 0)

    def tail(i, carry):
        j = sid + i * NS

        @pl.when(j < NCH_CORE)
        def _():
            base = cid * E_HALF + j * CH
            pltpu.sync_copy(dst_hbm.at[pl.ds(base, CH)], idx_v.at[0])
            pltpu.sync_copy(m_hbm.at[pl.ds(base, CH), pl.ds(0, MUL)],
                            rows_v.at[0])
            pltpu.sync_copy(rows_v.at[0], acc_sh.at[idx_v.at[0]], add=True)

        return carry

    lax.fori_loop(S_FULL * KB, S_TRIPS, tail, 0)
    plsc.subcore_barrier()
    pltpu.sync_copy(acc_sh.at[pl.ds(r0, ROWS_PER_TILE)],
                    out_hbm.at[pl.ds(cid * N + r0, ROWS_PER_TILE), pl.ds(0, MUL)])


_BLK = 8000


def _main_body(feat_ref, el_ref, ea_ref, w1_ref, w2_ref, o_ref):
    feat = feat_ref[...]
    el_t = el_ref[...].reshape(1, _BLK)
    ea_t = ea_ref[...].reshape(1, _BLK)
    xj = feat[:, 0:MUL]
    centers_t = lax.broadcasted_iota(
        jnp.int32, (NUM_RADIAL, 1), 0).astype(jnp.float32) * np.float32(5.0 / 7.0)
    d_t = el_t - centers_t
    radial_t = jnp.exp(-0.5 * d_t * d_t)
    w1 = w1_ref[...] * np.float32(1.0 / np.sqrt(NUM_RADIAL))
    h_t = jnp.dot(w1.T, radial_t, preferred_element_type=jnp.float32)
    h_t = h_t / (1.0 + jnp.exp(-h_t))
    h_t = h_t * (ea_t * np.float32(1.0 / np.sqrt(MUL)))
    h = jnp.transpose(h_t)
    w2 = w2_ref[...] * np.float32(1.0 / np.sqrt(HIDDEN))
    wts = jnp.dot(h, w2, preferred_element_type=jnp.float32)

    u_t = lax.broadcasted_iota(jnp.int32, (MUL, WEIGHT_NUMEL), 0)
    c_t = lax.broadcasted_iota(jnp.int32, (MUL, WEIGHT_NUMEL), 1)
    tile_m = jnp.where(c_t % MUL == u_t, 1.0, 0.0).astype(jnp.float32)
    xt = jnp.dot(xj, tile_m, preferred_element_type=jnp.float32)
    p = wts * xt
    r_s = lax.broadcasted_iota(jnp.int32, (WEIGHT_NUMEL, MUL), 0)
    w_s = lax.broadcasted_iota(jnp.int32, (WEIGHT_NUMEL, MUL), 1)
    seg_m = jnp.where(r_s // MUL == w_s, 1.0, 0.0).astype(jnp.float32)
    m = jnp.dot(p, seg_m, preferred_element_type=jnp.float32)
    o_ref[:, 0:MUL] = m


def _tc_main(feat, el3, ea3, W1, W2):
    grid = (E // _BLK,)
    return pl.pallas_call(
        _main_body,
        grid=grid,
        in_specs=[
            pl.BlockSpec((_BLK, 128), lambda i: (i, 0)),
            pl.BlockSpec((1, 1, _BLK), lambda i: (i, 0, 0)),
            pl.BlockSpec((1, 1, _BLK), lambda i: (i, 0, 0)),
            pl.BlockSpec((NUM_RADIAL, HIDDEN), lambda i: (0, 0)),
            pl.BlockSpec((HIDDEN, WEIGHT_NUMEL), lambda i: (0, 0)),
        ],
        out_specs=pl.BlockSpec((_BLK, 128), lambda i: (i, 0)),
        out_shape=jax.ShapeDtypeStruct((E, 128), jnp.float32),
    )(feat, el3, ea3, W1, W2)


def _comb_body(p_ref, x_ref, wsc_ref, o_ref):
    psum = p_ref[0:N, 0:MUL] + p_ref[N:2 * N, 0:MUL]
    wsc = wsc_ref[...] * np.float32(1.0 / np.sqrt(MUL))
    sc = jnp.dot(x_ref[...], wsc, preferred_element_type=jnp.float32)
    o_ref[...] = psum + sc


def _tc_combine(pfat, x, W_sc):
    return pl.pallas_call(
        _comb_body,
        out_shape=jax.ShapeDtypeStruct((N, MUL), jnp.float32),
    )(pfat, x, W_sc)


def kernel(x, edge_attr, edge_length, edge_src, edge_dst, W1, W2, W_sc):
    src = edge_src.astype(jnp.int32)
    dst = edge_dst.astype(jnp.int32)
    el3 = edge_length.reshape(E // _BLK, 1, _BLK)
    ea3 = edge_attr.reshape(E // _BLK, 1, _BLK)
    feat = _gather_k(x, src)
    mfat = _tc_main(feat, el3, ea3, W1, W2)
    zeros = jnp.zeros((N, MUL), dtype=jnp.float32)
    pfat = _scatter_k(mfat, dst, zeros)
    return _tc_combine(pfat, x, W_sc)
